# baseline (speedup 1.0000x reference)
import functools

import jax
import jax.numpy as jnp
import numpy as np
from jax import lax
from jax.experimental import pallas as pl
from jax.experimental.pallas import tpu as pltpu
from jax.experimental.pallas import tpu_sc as plsc

N = 10000
E = 320000
R = 8
D = 128

NC = 2
NS = 16
L = 16

NW = NC * NS
EPW = E // NW
CH = 40
NCHUNK = EPW // CH
CHW = 80
NB = 5

_PERM = np.empty((D,), np.int32)
for _k in range(D // 32):
  for _t in range(16):
    _PERM[32 * _k + 2 * _t] = 32 * _k + _t
    _PERM[32 * _k + 2 * _t + 1] = 32 * _k + 16 + _t

_MESH = plsc.VectorSubcoreMesh(
    core_axis_name="c", subcore_axis_name="s", num_cores=NC, num_subcores=NS)
_SC_PARAMS = pltpu.CompilerParams(
    use_tc_tiling_on_sc=False, needs_layout_passes=False)


EPW1 = E // NS
NCHUNK1 = EPW1 // CHW


@functools.partial(
    pl.kernel,
    out_type=jax.ShapeDtypeStruct((E,), jnp.float32),
    mesh=_MESH,
    compiler_params=_SC_PARAMS,
    scratch_types=[
        pltpu.VMEM_SHARED((N, L), jnp.float32),
        pltpu.VMEM((EPW1,), jnp.int32),
        pltpu.VMEM((EPW1,), jnp.int32),
        [pltpu.VMEM((CHW, L), jnp.float32)] * NB,
        [pltpu.VMEM((CHW, L), jnp.float32)] * NB,
        [pltpu.VMEM((CHW,), jnp.float32)] * NB,
        [pltpu.VMEM((CHW,), jnp.int32)] * NB,
        pltpu.VMEM((125, L), jnp.float32),
        pltpu.SemaphoreType.DMA((NB,)),
        pltpu.SemaphoreType.DMA((NB,)),
        pltpu.SemaphoreType.DMA((NB,)),
    ],
)
def _weights_kernel(dst_hbm, et_hbm, w_hbm,
                    cacc_sh, dst_v, et_v, oh, cr, wb, db, zero_v,
                    ssem, gsem, wsem):
  c = lax.axis_index("c")
  s = lax.axis_index("s")
  lanes = lax.iota(jnp.int32, L)
  ones16 = jnp.ones((L,), jnp.float32)
  zeros16 = jnp.zeros((L,), jnp.float32)

  e_base = s * EPW1
  pltpu.sync_copy(dst_hbm.at[pl.ds(e_base, EPW1)], dst_v)
  pltpu.sync_copy(et_hbm.at[pl.ds(e_base, EPW1)], et_v)

  @pl.loop(0, 125)
  def _(i):
    zero_v[i, :] = zeros16

  for j in range(5):
    pltpu.sync_copy(zero_v, cacc_sh.at[pl.ds(s * 625 + j * 125, 125)])

  for b in range(NB):
    @pl.loop(0, CHW)
    def _(i):
      oh[b][i, :] = zeros16

  plsc.subcore_barrier()

  @pl.loop(0, NCHUNK1 // NB)
  def _(g):
    for b in range(NB):
      i = g * NB + b

      @pl.when(i >= NB)
      def _():
        pltpu.make_async_copy(oh[b], cacc_sh.at[db[b]], ssem.at[b]).wait()
        for g16 in range(CHW // L):
          rid = lanes + g16 * L
          old = et_v[pl.ds((i - NB) * CHW + g16 * L, L)]
          plsc.store_scatter(oh[b], [rid, old], zeros16)

      for g16 in range(CHW // L):
        rid = lanes + g16 * L
        cur = et_v[pl.ds(i * CHW + g16 * L, L)]
        plsc.store_scatter(oh[b], [rid, cur], ones16)
        db[b][pl.ds(g16 * L, L)] = dst_v[pl.ds(i * CHW + g16 * L, L)]
      pltpu.async_copy(oh[b], cacc_sh.at[db[b]], ssem.at[b], add=True)

  for b in range(NB):
    pltpu.make_async_copy(oh[b], cacc_sh.at[db[b]], ssem.at[b]).wait()

  plsc.subcore_barrier()

  e2 = (c * NS + s) * EPW
  pltpu.sync_copy(dst_hbm.at[pl.ds(e2, EPW)], dst_v.at[pl.ds(0, EPW)])
  pltpu.sync_copy(et_hbm.at[pl.ds(e2, EPW)], et_v.at[pl.ds(0, EPW)])

  for b in range(2):
    pltpu.async_copy(cacc_sh.at[dst_v.at[pl.ds(b * CHW, CHW)]], cr[b],
                     gsem.at[b])

  @pl.loop(0, EPW // CHW // NB)
  def _(g):
    for b in range(NB):
      i = g * NB + b
      j = i + 2
      bj = (b + 2) % NB

      @pl.when(j < EPW // CHW)
      def _():
        pltpu.async_copy(cacc_sh.at[dst_v.at[pl.ds(j * CHW, CHW)]], cr[bj],
                         gsem.at[bj])

      pltpu.make_async_copy(cacc_sh.at[dst_v.at[pl.ds(i * CHW, CHW)]], cr[b],
                            gsem.at[b]).wait()

      @pl.when(i >= NB)
      def _():
        pltpu.make_async_copy(wb[b], w_hbm.at[pl.ds(0, CHW)],
                              wsem.at[b]).wait()

      for g16 in range(CHW // L):
        rid = lanes + g16 * L
        cur = et_v[pl.ds(i * CHW + g16 * L, L)]
        cnt = plsc.load_gather(cr[b], [rid, cur])
        wb[b][pl.ds(g16 * L, L)] = 1.0 / jnp.maximum(cnt, 1.0)
      pltpu.async_copy(wb[b], w_hbm.at[pl.ds(e2 + i * CHW, CHW)],
                       wsem.at[b])

  for b in range(NB):
    pltpu.make_async_copy(wb[b], w_hbm.at[pl.ds(0, CHW)], wsem.at[b]).wait()


@functools.partial(
    pl.kernel,
    out_type=jax.ShapeDtypeStruct((NC, N, D), jnp.float32),
    mesh=_MESH,
    compiler_params=_SC_PARAMS,
    scratch_types=[
        pltpu.VMEM_SHARED((N, D), jnp.float32),
        pltpu.VMEM((EPW,), jnp.int32),
        pltpu.VMEM((EPW,), jnp.float32),
        [pltpu.VMEM((CH, D), jnp.float32)] * NB,
        [pltpu.VMEM((CH,), jnp.int32)] * NB,
        pltpu.SemaphoreType.DMA((NB,)),
        pltpu.SemaphoreType.DMA((NB,)),
        pltpu.SemaphoreType.DMA((NB,)),
    ],
)
def _edge_agg_kernel(y_hbm, gidx_hbm, dst_hbm, w_hbm, accs_hbm,
                     acc_sh, gidx_v, w_v, rows, db,
                     gsem, ssem, dsem):
  c = lax.axis_index("c")
  s = lax.axis_index("s")
  slab = N // NS
  e_base = (c * NS + s) * EPW

  pltpu.sync_copy(gidx_hbm.at[pl.ds(e_base, EPW)], gidx_v)
  pltpu.sync_copy(w_hbm.at[pl.ds(e_base, EPW)], w_v)

  for b in range(4):
    pltpu.async_copy(dst_hbm.at[pl.ds(e_base + b * CH, CH)], db[b],
                     dsem.at[b])
    pltpu.async_copy(y_hbm.at[gidx_v.at[pl.ds(b * CH, CH)]], rows[b],
                     gsem.at[b])

  zv = jnp.zeros((L,), jnp.float32)

  @pl.loop(0, CH)
  def _(i):
    for k in range(D // L):
      rows[NB - 1][i, pl.ds(k * L, L)] = zv

  for j in range(15):
    pltpu.sync_copy(rows[NB - 1], acc_sh.at[pl.ds(s * slab + j * CH, CH)])
  pltpu.sync_copy(rows[NB - 1].at[pl.ds(0, slab - 15 * CH)],
                  acc_sh.at[pl.ds(s * slab + 15 * CH, slab - 15 * CH)])

  plsc.subcore_barrier()

  @pl.loop(0, NCHUNK // NB)
  def _(g):
    for b in range(NB):
      i = g * NB + b
      j = i + 4
      bj = (b + 4) % NB

      @pl.when(j < NCHUNK)
      def _():
        @pl.when(j >= NB)
        def _():
          pltpu.make_async_copy(rows[bj], acc_sh.at[db[bj]],
                                ssem.at[bj]).wait()
        pltpu.async_copy(dst_hbm.at[pl.ds(e_base + j * CH, CH)], db[bj],
                         dsem.at[bj])
        pltpu.async_copy(y_hbm.at[gidx_v.at[pl.ds(j * CH, CH)]], rows[bj],
                         gsem.at[bj])

      pltpu.make_async_copy(y_hbm.at[gidx_v.at[pl.ds(i * CH, CH)]], rows[b],
                            gsem.at[b]).wait()

      @pl.loop(0, CH, unroll=2)
      def _(e):
        wspl = plsc.load_gather(
            w_v, [jnp.broadcast_to(i * CH + e, (L,)).astype(jnp.int32)])
        for k in range(D // L):
          rows[b][e, pl.ds(k * L, L)] = rows[b][e, pl.ds(k * L, L)] * wspl

      pltpu.make_async_copy(dst_hbm.at[pl.ds(0, CH)], db[b],
                            dsem.at[b]).wait()
      pltpu.async_copy(rows[b], acc_sh.at[db[b]], ssem.at[b], add=True)

  for b in range(NB):
    pltpu.make_async_copy(rows[b], acc_sh.at[db[b]], ssem.at[b]).wait()

  plsc.subcore_barrier()
  pltpu.sync_copy(acc_sh.at[pl.ds(s * slab, slab)],
                  accs_hbm.at[c, pl.ds(s * slab, slab)])


_BM = 1000


def _mm(h, w):
  return jnp.dot(h, w, preferred_element_type=jnp.float32)


def _layer1_body(x_ref, w_ref, root_ref, b_ref, y_ref, z_ref):
  hb = x_ref[...].astype(jnp.bfloat16)
  for r in range(R):
    y_ref[r] = _mm(hb, w_ref[r])
  z_ref[...] = _mm(hb, root_ref[...]) + b_ref[...]


def _layer_body(z_ref, acc_ref, w_ref, root_ref, b_ref, y_ref, zo_ref):
  h = jnp.maximum(z_ref[...] + acc_ref[0] + acc_ref[1], 0.0)
  hb = h.astype(jnp.bfloat16)
  for r in range(R):
    y_ref[r] = _mm(hb, w_ref[r])
  zo_ref[...] = _mm(hb, root_ref[...]) + b_ref[...]


_YZ_OUT = [
    jax.ShapeDtypeStruct((R, N, D), jnp.float32),
    jax.ShapeDtypeStruct((N, D), jnp.float32),
]
_YZ_SPECS = dict(
    grid=(N // _BM,),
    out_specs=[
        pl.BlockSpec((R, _BM, D), lambda i: (0, i, 0)),
        pl.BlockSpec((_BM, D), lambda i: (i, 0)),
    ],
    out_shape=_YZ_OUT,
)
_W_SPECS = [
    pl.BlockSpec((R, D, D), lambda i: (0, 0, 0)),
    pl.BlockSpec((D, D), lambda i: (0, 0)),
    pl.BlockSpec((1, D), lambda i: (0, 0)),
]


def _layer1(x, W, root, b):
  return pl.pallas_call(
      _layer1_body,
      in_specs=[pl.BlockSpec((_BM, D), lambda i: (i, 0))] + _W_SPECS,
      **_YZ_SPECS,
  )(x, W, root, b)


def _layer(z, accs, W, root, b):
  return pl.pallas_call(
      _layer_body,
      in_specs=[
          pl.BlockSpec((_BM, D), lambda i: (i, 0)),
          pl.BlockSpec((NC, _BM, D), lambda i: (0, i, 0)),
      ] + _W_SPECS,
      **_YZ_SPECS,
  )(z, accs, W, root, b)


def _tail_body(z_ref, acc_ref, o_ref):
  o_ref[...] = z_ref[...] + acc_ref[0] + acc_ref[1]


def _tail(z, accs):
  return pl.pallas_call(
      _tail_body,
      grid=(N // _BM,),
      in_specs=[
          pl.BlockSpec((_BM, D), lambda i: (i, 0)),
          pl.BlockSpec((NC, _BM, D), lambda i: (0, i, 0)),
      ],
      out_specs=pl.BlockSpec((_BM, D), lambda i: (i, 0)),
      out_shape=jax.ShapeDtypeStruct((N, D), jnp.float32),
  )(z, accs)


def kernel(x, edge_index, edge_type, W1, root1, b1, W2, root2, b2,
           W3, root3, b3):
  src = edge_index[0]
  dst = edge_index[1]
  et = edge_type
  gidx = et * N + src

  w = _weights_kernel(dst, et)

  bf = jnp.bfloat16
  yt, z = _layer1(x, W1.astype(bf), root1.astype(bf), b1.reshape(1, D))
  accs = _edge_agg_kernel(yt.reshape(R * N, D), gidx, dst, w)
  for Wl, rootl, bl in ((W2, root2, b2), (W3, root3, b3)):
    yt, z = _layer(z, accs, Wl.astype(bf), rootl.astype(bf),
                   bl.reshape(1, D))
    accs = _edge_agg_kernel(yt.reshape(R * N, D), gidx, dst, w)
  return _tail(z, accs)


# speedup vs baseline: 1.3318x; 1.3318x over previous
import functools

import jax
import jax.numpy as jnp
import numpy as np
from jax import lax
from jax.experimental import pallas as pl
from jax.experimental.pallas import tpu as pltpu
from jax.experimental.pallas import tpu_sc as plsc

N = 10000
E = 320000
R = 8
D = 128

NC = 2
NS = 16
L = 16

NW = NC * NS
EPW = E // NW
CH = 40
NCHUNK = EPW // CH
CHW = 80
NB = 5

_PERM = np.empty((D,), np.int32)
for _k in range(D // 32):
  for _t in range(16):
    _PERM[32 * _k + 2 * _t] = 32 * _k + _t
    _PERM[32 * _k + 2 * _t + 1] = 32 * _k + 16 + _t

_MESH = plsc.VectorSubcoreMesh(
    core_axis_name="c", subcore_axis_name="s", num_cores=NC, num_subcores=NS)
_SC_PARAMS = pltpu.CompilerParams(
    use_tc_tiling_on_sc=False, needs_layout_passes=False)


EPW1 = E // NS
NCHUNK1 = EPW1 // CHW


@functools.partial(
    pl.kernel,
    out_type=jax.ShapeDtypeStruct((E,), jnp.float32),
    mesh=_MESH,
    compiler_params=_SC_PARAMS,
    scratch_types=[
        pltpu.VMEM_SHARED((N, L), jnp.float32),
        pltpu.VMEM((EPW1,), jnp.int32),
        pltpu.VMEM((EPW1,), jnp.int32),
        [pltpu.VMEM((CHW, L), jnp.float32)] * NB,
        [pltpu.VMEM((CHW, L), jnp.float32)] * NB,
        [pltpu.VMEM((CHW,), jnp.float32)] * NB,
        [pltpu.VMEM((CHW,), jnp.int32)] * NB,
        pltpu.VMEM((125, L), jnp.float32),
        pltpu.SemaphoreType.DMA((NB,)),
        pltpu.SemaphoreType.DMA((NB,)),
        pltpu.SemaphoreType.DMA((NB,)),
    ],
)
def _weights_kernel(dst_hbm, et_hbm, w_hbm,
                    cacc_sh, dst_v, et_v, oh, cr, wb, db, zero_v,
                    ssem, gsem, wsem):
  c = lax.axis_index("c")
  s = lax.axis_index("s")
  lanes = lax.iota(jnp.int32, L)
  ones16 = jnp.ones((L,), jnp.float32)
  zeros16 = jnp.zeros((L,), jnp.float32)

  e_base = s * EPW1
  pltpu.sync_copy(dst_hbm.at[pl.ds(e_base, EPW1)], dst_v)
  pltpu.sync_copy(et_hbm.at[pl.ds(e_base, EPW1)], et_v)

  @pl.loop(0, 125)
  def _(i):
    zero_v[i, :] = zeros16

  for j in range(5):
    pltpu.sync_copy(zero_v, cacc_sh.at[pl.ds(s * 625 + j * 125, 125)])

  for b in range(NB):
    @pl.loop(0, CHW)
    def _(i):
      oh[b][i, :] = zeros16

  plsc.subcore_barrier()

  @pl.loop(0, NCHUNK1 // NB)
  def _(g):
    for b in range(NB):
      i = g * NB + b

      @pl.when(i >= NB)
      def _():
        pltpu.make_async_copy(oh[b], cacc_sh.at[db[b]], ssem.at[b]).wait()
        for g16 in range(CHW // L):
          rid = lanes + g16 * L
          old = et_v[pl.ds((i - NB) * CHW + g16 * L, L)]
          plsc.store_scatter(oh[b], [rid, old], zeros16)

      for g16 in range(CHW // L):
        rid = lanes + g16 * L
        cur = et_v[pl.ds(i * CHW + g16 * L, L)]
        plsc.store_scatter(oh[b], [rid, cur], ones16)
        db[b][pl.ds(g16 * L, L)] = dst_v[pl.ds(i * CHW + g16 * L, L)]
      pltpu.async_copy(oh[b], cacc_sh.at[db[b]], ssem.at[b], add=True)

  for b in range(NB):
    pltpu.make_async_copy(oh[b], cacc_sh.at[db[b]], ssem.at[b]).wait()

  plsc.subcore_barrier()

  e2 = (c * NS + s) * EPW
  pltpu.sync_copy(dst_hbm.at[pl.ds(e2, EPW)], dst_v.at[pl.ds(0, EPW)])
  pltpu.sync_copy(et_hbm.at[pl.ds(e2, EPW)], et_v.at[pl.ds(0, EPW)])

  for b in range(2):
    pltpu.async_copy(cacc_sh.at[dst_v.at[pl.ds(b * CHW, CHW)]], cr[b],
                     gsem.at[b])

  @pl.loop(0, EPW // CHW // NB)
  def _(g):
    for b in range(NB):
      i = g * NB + b
      j = i + 2
      bj = (b + 2) % NB

      @pl.when(j < EPW // CHW)
      def _():
        pltpu.async_copy(cacc_sh.at[dst_v.at[pl.ds(j * CHW, CHW)]], cr[bj],
                         gsem.at[bj])

      pltpu.make_async_copy(cacc_sh.at[dst_v.at[pl.ds(i * CHW, CHW)]], cr[b],
                            gsem.at[b]).wait()

      @pl.when(i >= NB)
      def _():
        pltpu.make_async_copy(wb[b], w_hbm.at[pl.ds(0, CHW)],
                              wsem.at[b]).wait()

      for g16 in range(CHW // L):
        rid = lanes + g16 * L
        cur = et_v[pl.ds(i * CHW + g16 * L, L)]
        cnt = plsc.load_gather(cr[b], [rid, cur])
        wb[b][pl.ds(g16 * L, L)] = 1.0 / jnp.maximum(cnt, 1.0)
      pltpu.async_copy(wb[b], w_hbm.at[pl.ds(e2 + i * CHW, CHW)],
                       wsem.at[b])

  for b in range(NB):
    pltpu.make_async_copy(wb[b], w_hbm.at[pl.ds(0, CHW)], wsem.at[b]).wait()


@functools.partial(
    pl.kernel,
    out_type=jax.ShapeDtypeStruct((NC, N, D), jnp.float32),
    mesh=_MESH,
    compiler_params=_SC_PARAMS,
    scratch_types=[
        pltpu.VMEM_SHARED((N, D), jnp.float32),
        pltpu.VMEM((EPW,), jnp.int32),
        pltpu.VMEM((EPW,), jnp.float32),
        [pltpu.VMEM((CH, D), jnp.float32)] * NB,
        [pltpu.VMEM((CH,), jnp.int32)] * NB,
        pltpu.SemaphoreType.DMA((NB,)),
        pltpu.SemaphoreType.DMA((NB,)),
        pltpu.SemaphoreType.DMA((NB,)),
    ],
)
def _edge_agg_kernel(y_hbm, gidx_hbm, dst_hbm, w_hbm, accs_hbm,
                     acc_sh, gidx_v, w_v, rows, db,
                     gsem, ssem, dsem):
  c = lax.axis_index("c")
  s = lax.axis_index("s")
  slab = N // NS
  e_base = (c * NS + s) * EPW

  pltpu.sync_copy(gidx_hbm.at[pl.ds(e_base, EPW)], gidx_v)
  pltpu.sync_copy(w_hbm.at[pl.ds(e_base, EPW)], w_v)

  for b in range(3):
    pltpu.async_copy(dst_hbm.at[pl.ds(e_base + b * CH, CH)], db[b],
                     dsem.at[b])
    pltpu.async_copy(y_hbm.at[gidx_v.at[pl.ds(b * CH, CH)]], rows[b],
                     gsem.at[b])

  zv = jnp.zeros((L,), jnp.float32)

  @pl.loop(0, CH)
  def _(i):
    for k in range(D // L):
      rows[NB - 1][i, pl.ds(k * L, L)] = zv

  for j in range(15):
    pltpu.sync_copy(rows[NB - 1], acc_sh.at[pl.ds(s * slab + j * CH, CH)])
  pltpu.sync_copy(rows[NB - 1].at[pl.ds(0, slab - 15 * CH)],
                  acc_sh.at[pl.ds(s * slab + 15 * CH, slab - 15 * CH)])

  plsc.subcore_barrier()

  @pl.loop(0, NCHUNK // NB)
  def _(g):
    for b in range(NB):
      i = g * NB + b
      j = i + 3
      bj = (b + 3) % NB

      @pl.when(j < NCHUNK)
      def _():
        @pl.when(j >= NB)
        def _():
          pltpu.make_async_copy(rows[bj], acc_sh.at[db[bj]],
                                ssem.at[bj]).wait()
        pltpu.async_copy(dst_hbm.at[pl.ds(e_base + j * CH, CH)], db[bj],
                         dsem.at[bj])
        pltpu.async_copy(y_hbm.at[gidx_v.at[pl.ds(j * CH, CH)]], rows[bj],
                         gsem.at[bj])

      pltpu.make_async_copy(y_hbm.at[gidx_v.at[pl.ds(i * CH, CH)]], rows[b],
                            gsem.at[b]).wait()

      @pl.loop(0, CH, unroll=2)
      def _(e):
        wspl = plsc.load_gather(
            w_v, [jnp.broadcast_to(i * CH + e, (L,)).astype(jnp.int32)])
        for k in range(D // L):
          rows[b][e, pl.ds(k * L, L)] = rows[b][e, pl.ds(k * L, L)] * wspl

      pltpu.make_async_copy(dst_hbm.at[pl.ds(0, CH)], db[b],
                            dsem.at[b]).wait()
      pltpu.async_copy(rows[b], acc_sh.at[db[b]], ssem.at[b], add=True)

  for b in range(NB):
    pltpu.make_async_copy(rows[b], acc_sh.at[db[b]], ssem.at[b]).wait()

  plsc.subcore_barrier()
  pltpu.sync_copy(acc_sh.at[pl.ds(s * slab, slab)],
                  accs_hbm.at[c, pl.ds(s * slab, slab)])


_BM = 1000


def _mm(h, w):
  return jnp.dot(h, w, preferred_element_type=jnp.float32)


def _layer1_body(x_ref, w_ref, root_ref, b_ref, y_ref, z_ref):
  hb = x_ref[...].astype(jnp.bfloat16)
  for r in range(R):
    y_ref[r] = _mm(hb, w_ref[r])
  z_ref[...] = _mm(hb, root_ref[...]) + b_ref[...]


def _layer_body(z_ref, acc_ref, w_ref, root_ref, b_ref, y_ref, zo_ref):
  h = jnp.maximum(z_ref[...] + acc_ref[0] + acc_ref[1], 0.0)
  hb = h.astype(jnp.bfloat16)
  for r in range(R):
    y_ref[r] = _mm(hb, w_ref[r])
  zo_ref[...] = _mm(hb, root_ref[...]) + b_ref[...]


_YZ_OUT = [
    jax.ShapeDtypeStruct((R, N, D), jnp.float32),
    jax.ShapeDtypeStruct((N, D), jnp.float32),
]
_YZ_SPECS = dict(
    grid=(N // _BM,),
    out_specs=[
        pl.BlockSpec((R, _BM, D), lambda i: (0, i, 0)),
        pl.BlockSpec((_BM, D), lambda i: (i, 0)),
    ],
    out_shape=_YZ_OUT,
)
_W_SPECS = [
    pl.BlockSpec((R, D, D), lambda i: (0, 0, 0)),
    pl.BlockSpec((D, D), lambda i: (0, 0)),
    pl.BlockSpec((1, D), lambda i: (0, 0)),
]


def _layer1(x, W, root, b):
  return pl.pallas_call(
      _layer1_body,
      in_specs=[pl.BlockSpec((_BM, D), lambda i: (i, 0))] + _W_SPECS,
      **_YZ_SPECS,
  )(x, W, root, b)


def _layer(z, accs, W, root, b):
  return pl.pallas_call(
      _layer_body,
      in_specs=[
          pl.BlockSpec((_BM, D), lambda i: (i, 0)),
          pl.BlockSpec((NC, _BM, D), lambda i: (0, i, 0)),
      ] + _W_SPECS,
      **_YZ_SPECS,
  )(z, accs, W, root, b)


def _tail_body(z_ref, acc_ref, o_ref):
  o_ref[...] = z_ref[...] + acc_ref[0] + acc_ref[1]


def _tail(z, accs):
  return pl.pallas_call(
      _tail_body,
      grid=(N // _BM,),
      in_specs=[
          pl.BlockSpec((_BM, D), lambda i: (i, 0)),
          pl.BlockSpec((NC, _BM, D), lambda i: (0, i, 0)),
      ],
      out_specs=pl.BlockSpec((_BM, D), lambda i: (i, 0)),
      out_shape=jax.ShapeDtypeStruct((N, D), jnp.float32),
  )(z, accs)


def kernel(x, edge_index, edge_type, W1, root1, b1, W2, root2, b2,
           W3, root3, b3):
  src = edge_index[0]
  dst = edge_index[1]
  et = edge_type
  gidx = et * N + src

  w = _weights_kernel(dst, et)

  bf = jnp.bfloat16
  yt, z = _layer1(x, W1.astype(bf), root1.astype(bf), b1.reshape(1, D))
  accs = _edge_agg_kernel(yt.reshape(R * N, D), gidx, dst, w)
  for Wl, rootl, bl in ((W2, root2, b2), (W3, root3, b3)):
    yt, z = _layer(z, accs, Wl.astype(bf), rootl.astype(bf),
                   bl.reshape(1, D))
    accs = _edge_agg_kernel(yt.reshape(R * N, D), gidx, dst, w)
  return _tail(z, accs)
